# Initial kernel scaffold; baseline (speedup 1.0000x reference)
import jax
import jax.numpy as jnp
from jax.experimental import pallas as pl


def kernel(w1, b1, w2, b2, w3, b3, wfc1, bfc1, wfc2, bfc2, x):
    raise NotImplementedError("write your pallas kernel here")


# speedup vs baseline: 1.9727x; 1.9727x over previous
from functools import partial

import jax
import jax.numpy as jnp
from jax.experimental import pallas as pl
from jax.experimental.pallas import tpu as pltpu

_B = 16


def _fused_cnn_kernel(x_ref, w1_ref, b1_ref, w2_ref, b2_ref, w3_ref, b3_ref,
                      wfc1_ref, bfc1_ref, wfc2_ref, bfc2_ref, o_ref,
                      s1, s2in, s2, s3in, s3):
    B = _B
    f32 = jnp.float32
    bf16 = jnp.bfloat16

    def conv9(src_ref, w_ref, b_ref, L, Wp, Cin, Cout):
        acc = jnp.zeros((B * L, Cout), f32)
        for ky in range(3):
            for kx in range(3):
                patch = src_ref[:, pl.ds(ky * Wp + kx, L), :]
                patch = patch.reshape(B * L, Cin).astype(bf16)
                acc = acc + jnp.dot(patch, w_ref[ky * 3 + kx],
                                    preferred_element_type=f32)
        return jnp.maximum(acc + b_ref[...], 0.0)

    def pool_rows(s_ref, i, Wp, Wo):
        r0 = 2 * i * Wp
        a = s_ref[:, pl.ds(r0, Wo, stride=2), :]
        b = s_ref[:, pl.ds(r0 + 1, Wo, stride=2), :]
        c = s_ref[:, pl.ds(r0 + Wp, Wo, stride=2), :]
        d = s_ref[:, pl.ds(r0 + Wp + 1, Wo, stride=2), :]
        return jnp.maximum(jnp.maximum(a, b), jnp.maximum(c, d))

    s1[...] = conv9(x_ref, w1_ref, b1_ref, 840, 30, 8, 32).reshape(B, 840, 32)
    s2in[...] = jnp.zeros((B, 17 * 16, 32), f32)
    for i in range(14):
        s2in[:, pl.ds((i + 1) * 16 + 1, 14), :] = pool_rows(s1, i, 30, 14)

    s2[...] = conv9(s2in, w2_ref, b2_ref, 224, 16, 32, 64).reshape(B, 224, 64)
    s3in[...] = jnp.zeros((B, 11 * 10, 64), f32)
    for i in range(7):
        s3in[:, pl.ds((i + 2) * 10 + 2, 7), :] = pool_rows(s2, i, 16, 7)

    y3 = conv9(s3in, w3_ref, b3_ref, 80, 10, 64, 128)
    r = jax.lax.broadcasted_iota(jnp.int32, (B * 80, 128), 0)
    l = r % 80
    y3 = jnp.where(jnp.logical_and(l % 10 >= 1, l >= 10), y3, 0.0)
    s3[...] = y3.reshape(B, 80, 128)

    acc1 = jnp.zeros((B, 640), f32)
    for i in range(4):
        m = pool_rows(s3, i, 10, 4)
        for w in range(4):
            acc1 = acc1 + jnp.dot(m[:, w, :].astype(bf16),
                                  wfc1_ref[i * 4 + w],
                                  preferred_element_type=f32)
    h1 = jnp.maximum(acc1 + bfc1_ref[...], 0.0).astype(bf16)

    out = jnp.dot(h1, wfc2_ref[...], preferred_element_type=f32)
    o_ref[...] = out + bfc2_ref[...]


def kernel(w1, b1, w2, b2, w3, b3, wfc1, bfc1, wfc2, bfc2, x):
    N = x.shape[0]
    B = _B
    Np = ((N + B - 1) // B) * B

    xs = jnp.transpose(x, (0, 2, 3, 1)).astype(jnp.float32)
    xs = jnp.pad(xs, ((0, Np - N), (1, 2), (1, 1), (0, 7)))
    xs = xs.reshape(Np, 31 * 30, 8)

    wfc1r = wfc1.reshape(128, 16, 640).transpose(1, 0, 2).astype(jnp.bfloat16)

    full = lambda shape: pl.BlockSpec(shape, lambda n: (0,) * len(shape))

    out = pl.pallas_call(
        _fused_cnn_kernel,
        out_shape=jax.ShapeDtypeStruct((Np, 128), jnp.float32),
        grid=(Np // B,),
        in_specs=[
            pl.BlockSpec((B, 31 * 30, 8), lambda n: (n, 0, 0)),
            full((9, 8, 32)), full((1, 32)),
            full((9, 32, 64)), full((1, 64)),
            full((9, 64, 128)), full((1, 128)),
            full((16, 128, 640)), full((1, 640)),
            full((640, 128)), full((1, 128)),
        ],
        out_specs=pl.BlockSpec((B, 128), lambda n: (n, 0)),
        scratch_shapes=[
            pltpu.VMEM((B, 840, 32), jnp.float32),
            pltpu.VMEM((B, 272, 32), jnp.float32),
            pltpu.VMEM((B, 224, 64), jnp.float32),
            pltpu.VMEM((B, 110, 64), jnp.float32),
            pltpu.VMEM((B, 80, 128), jnp.float32),
        ],
        compiler_params=pltpu.CompilerParams(dimension_semantics=("parallel",)),
    )(xs, w1.astype(jnp.bfloat16), b1.reshape(1, 32).astype(jnp.float32),
      w2.astype(jnp.bfloat16), b2.reshape(1, 64).astype(jnp.float32),
      w3.astype(jnp.bfloat16), b3.reshape(1, 128).astype(jnp.float32),
      wfc1r, bfc1.reshape(1, 640).astype(jnp.float32),
      wfc2.astype(jnp.bfloat16), bfc2.reshape(1, 128).astype(jnp.float32))

    return out[:N, :10]
